# baseline (speedup 1.0000x reference)
import functools

import jax
import jax.numpy as jnp
import numpy as np
from jax import lax
from jax.experimental import pallas as pl
from jax.experimental.pallas import tpu as pltpu
from jax.experimental.pallas import tpu_sc as plsc

G = 4
_BF = jnp.bfloat16
_F32 = jnp.float32


def _silu(v):
    return v * jax.nn.sigmoid(v)


def _split3(v):
    h1 = v.astype(_BF)
    r1 = v - h1.astype(_F32)
    h2 = r1.astype(_BF)
    h3 = (r1 - h2.astype(_F32)).astype(_BF)
    return h1, h2, h3


def _dotx(oh_bf, v):
    h1, h2, h3 = _split3(v)
    return (jnp.dot(oh_bf, h1, preferred_element_type=_F32)
            + jnp.dot(oh_bf, h2, preferred_element_type=_F32)
            + jnp.dot(oh_bf, h3, preferred_element_type=_F32))


def _dotx_pre(oh_bf, parts):
    return (jnp.dot(oh_bf, parts[0], preferred_element_type=_F32)
            + jnp.dot(oh_bf, parts[1], preferred_element_type=_F32)
            + jnp.dot(oh_bf, parts[2], preferred_element_type=_F32))


def _dotb(a, b_bf):
    return jnp.dot(a.astype(_BF), b_bf, preferred_element_type=_F32)


def _sc_embed_gather(a_idx, atom_emb):
    n = a_idx.shape[0]
    d = atom_emb.shape[1]
    info = plsc.get_sparse_core_info()
    nw = info.num_cores * info.num_subcores
    ch = 128
    per_w = n // nw
    mesh = plsc.VectorSubcoreMesh(core_axis_name="c", subcore_axis_name="s")

    @functools.partial(
        pl.kernel, mesh=mesh,
        out_type=jax.ShapeDtypeStruct((n, d), jnp.float32),
        scratch_types=[
            pltpu.VMEM((ch,), jnp.int32),
            pltpu.VMEM((ch, d), jnp.float32),
            pltpu.SemaphoreType.DMA,
        ],
    )
    def k(idx_hbm, table_hbm, out_hbm, idx_v, rows_v, sem):
        wid = lax.axis_index("s") * info.num_cores + lax.axis_index("c")
        base = wid * per_w
        for c in range(per_w // ch):
            off = base + c * ch
            pltpu.sync_copy(idx_hbm.at[pl.ds(off, ch)], idx_v)
            pltpu.async_copy(table_hbm.at[idx_v], rows_v, sem).wait()
            pltpu.sync_copy(rows_v, out_hbm.at[pl.ds(off, ch)])

    return k(a_idx, atom_emb)


def _make_body(aps, epb, d, nrbf, nelem):
    del nelem
    def body(h0_ref, t_ref, l_ref, x_ref, src_ref, dst_ref,
             wt_ref, wlat_ref, wrbf_ref, w1a_ref, w1b_ref,
             w2a_ref, w2b_ref, wg_ref, we_ref,
             posv_ref, pe_ref):
        half = d // 2
        freqs = jnp.exp(
            lax.broadcasted_iota(jnp.int32, (1, half), 1).astype(_F32)
            * (-np.log(10000.0) / half))
        tv = t_ref[0].astype(_F32)
        args = tv * freqs
        temb = jnp.concatenate([jnp.sin(args), jnp.cos(args)], axis=-1)
        wt_bf = wt_ref[...].astype(_BF)
        wlat_bf = wlat_ref[...].astype(_BF)
        base = _dotb(temb, wt_bf) + _dotb(l_ref[0], wlat_bf)
        cent = (lax.broadcasted_iota(jnp.int32, (1, nrbf), 1).astype(_F32)
                * (12.0 / (nrbf - 1)))
        wrbf_bf = wrbf_ref[...].astype(_BF)
        w1a_bf = w1a_ref[...].astype(_BF)
        w1b_bf = w1b_ref[...].astype(_BF)
        w2a_bf = w2a_ref[...].astype(_BF)
        w2b_bf = w2b_ref[...].astype(_BF)
        wg_bf = wg_ref[...].astype(_BF)
        we_bf = we_ref[...].astype(_BF)

        for j in range(G):
            h = (h0_ref[0, j * aps:(j + 1) * aps, :]
                 + base[j:j + 1])
            srcl = src_ref[0, j] & (aps - 1)
            dstl = dst_ref[0, j] & (aps - 1)
            eiota = lax.broadcasted_iota(jnp.int32, (epb, aps), 1)
            Pm = eiota == srcl[:, None]
            Qm = eiota == dstl[:, None]
            P = Pm.astype(_BF)
            R = (Qm.astype(_F32) - Pm.astype(_F32)).astype(_BF)
            Qt = (lax.broadcasted_iota(jnp.int32, (aps, epb), 0)
                  == dstl[None, :]).astype(_BF)
            xs = x_ref[0, j * aps:(j + 1) * aps, :]
            vec = _dotx(R, xs)
            d2 = jnp.sum(vec * vec, axis=-1, keepdims=True)
            dist = jnp.sqrt(d2 + 1e-12)
            dirn = vec / (dist + 1e-8)
            rbf = jnp.exp(-((dist - cent) ** 2) * 2.0)
            ebf = _dotb(rbf, wrbf_bf)
            s1 = _silu(_dotb(h, w1a_bf))
            m = _dotx(P, s1) * ebf
            agg = _dotx(Qt, m)
            h = h + _silu(_dotb(agg, w1b_bf))
            s2 = _silu(_dotb(h, w2a_bf))
            m = _dotx(P, s2) * ebf
            agg = _dotx(Qt, m)
            h = h + _silu(_dotb(agg, w2b_bf))
            sg = _silu(h)
            ge = _dotx(P, sg) * ebf
            gate = _dotb(ge, wg_bf)
            posv_ref[0, j * aps:(j + 1) * aps, :] = _dotx(Qt, dirn * gate)
            ea = _dotb(h, we_bf)
            pe_ref[0, j, :] = jnp.sum(ea, axis=0)
    return body


def kernel(a, l, x, n, t, edge_index, atom_emb, W_t, W_lat, W_rbf,
           W1a, W1b, W2a, W2b, W_gate, W_energy):
    del n
    b = t.shape[0]
    natoms = a.shape[0]
    aps = natoms // b
    e = edge_index.shape[1]
    epb = e // b
    d = atom_emb.shape[1]
    nrbf = W_rbf.shape[0]
    nelem = atom_emb.shape[0]
    nb = b // G

    h0 = _sc_embed_gather(a.astype(jnp.int32), atom_emb)
    h03 = h0.reshape(nb, G * aps, d)
    t3 = t.reshape(nb, G, 1).astype(jnp.int32)
    l3 = l.reshape(b, 9).reshape(nb, G, 9)
    x3 = x.reshape(nb, G * aps, 3)
    src3 = edge_index[0].reshape(nb, G, epb)
    dst3 = edge_index[1].reshape(nb, G, epb)

    full = lambda shape: pl.BlockSpec(shape, lambda i: tuple(0 for _ in shape))
    posv, pe = pl.pallas_call(
        _make_body(aps, epb, d, nrbf, nelem),
        grid=(nb,),
        in_specs=[
            pl.BlockSpec((1, G * aps, d), lambda i: (i, 0, 0)),
            pl.BlockSpec((1, G, 1), lambda i: (i, 0, 0)),
            pl.BlockSpec((1, G, 9), lambda i: (i, 0, 0)),
            pl.BlockSpec((1, G * aps, 3), lambda i: (i, 0, 0)),
            pl.BlockSpec((1, G, epb), lambda i: (i, 0, 0)),
            pl.BlockSpec((1, G, epb), lambda i: (i, 0, 0)),
            full((d, d)),
            full((9, d)),
            full((nrbf, d)),
            full((d, d)),
            full((d, d)),
            full((d, d)),
            full((d, d)),
            full((d, 1)),
            full((d, 1)),
        ],
        out_specs=[
            pl.BlockSpec((1, G * aps, 3), lambda i: (i, 0, 0)),
            pl.BlockSpec((1, G, 1), lambda i: (i, 0, 0)),
        ],
        out_shape=[
            jax.ShapeDtypeStruct((nb, G * aps, 3), jnp.float32),
            jax.ShapeDtypeStruct((nb, G, 1), jnp.float32),
        ],
    )(h03, t3, l3, x3, src3, dst3, W_t, W_lat, W_rbf,
      W1a, W1b, W2a, W2b, W_gate, W_energy)
    return posv.reshape(natoms, 3), pe.reshape(b)


# speedup vs baseline: 2.4142x; 2.4142x over previous
import functools

import jax
import jax.numpy as jnp
import numpy as np
from jax import lax
from jax.experimental import pallas as pl
from jax.experimental.pallas import tpu as pltpu
from jax.experimental.pallas import tpu_sc as plsc

G = 4
_BF = jnp.bfloat16
_F32 = jnp.float32


def _silu(v):
    return v * jax.nn.sigmoid(v)


def _split3(v):
    h1 = v.astype(_BF)
    r1 = v - h1.astype(_F32)
    h2 = r1.astype(_BF)
    h3 = (r1 - h2.astype(_F32)).astype(_BF)
    return h1, h2, h3


def _dotx(oh_bf, v):
    h1, h2, h3 = _split3(v)
    return (jnp.dot(oh_bf, h1, preferred_element_type=_F32)
            + jnp.dot(oh_bf, h2, preferred_element_type=_F32)
            + jnp.dot(oh_bf, h3, preferred_element_type=_F32))


def _dot2(oh_bf, v):
    h1 = v.astype(_BF)
    h2 = (v - h1.astype(_F32)).astype(_BF)
    return (jnp.dot(oh_bf, h1, preferred_element_type=_F32)
            + jnp.dot(oh_bf, h2, preferred_element_type=_F32))


def _dotb(a, b_bf):
    return jnp.dot(a.astype(_BF), b_bf, preferred_element_type=_F32)


def _sc_embed_gather(a_idx, atom_emb):
    n = a_idx.shape[0]
    d = atom_emb.shape[1]
    info = plsc.get_sparse_core_info()
    nw = info.num_cores * info.num_subcores
    ch = 128
    per_w = n // nw
    mesh = plsc.VectorSubcoreMesh(core_axis_name="c", subcore_axis_name="s")

    @functools.partial(
        pl.kernel, mesh=mesh,
        out_type=jax.ShapeDtypeStruct((n, d), jnp.float32),
        scratch_types=[
            pltpu.VMEM((ch,), jnp.int32),
            pltpu.VMEM((ch, d), jnp.float32),
            pltpu.SemaphoreType.DMA,
        ],
    )
    def k(idx_hbm, table_hbm, out_hbm, idx_v, rows_v, sem):
        wid = lax.axis_index("s") * info.num_cores + lax.axis_index("c")
        base = wid * per_w
        for c in range(per_w // ch):
            off = base + c * ch
            pltpu.sync_copy(idx_hbm.at[pl.ds(off, ch)], idx_v)
            pltpu.async_copy(table_hbm.at[idx_v], rows_v, sem).wait()
            pltpu.sync_copy(rows_v, out_hbm.at[pl.ds(off, ch)])

    return k(a_idx, atom_emb)


def _make_body(aps, epb, d, nrbf, nelem):
    del nelem
    def body(h0_ref, t_ref, l_ref, x_ref, src_ref, dst_ref,
             wt_ref, wlat_ref, wrbf_ref, w1a_ref, w1b_ref,
             w2a_ref, w2b_ref, wg_ref, we_ref,
             posv_ref, pe_ref):
        half = d // 2
        freqs = jnp.exp(
            lax.broadcasted_iota(jnp.int32, (1, half), 1).astype(_F32)
            * (-np.log(10000.0) / half))
        tv = t_ref[0].astype(_F32)
        args = tv * freqs
        temb = jnp.concatenate([jnp.sin(args), jnp.cos(args)], axis=-1)
        wt_bf = wt_ref[...].astype(_BF)
        wlat_bf = wlat_ref[...].astype(_BF)
        base = _dotb(temb, wt_bf) + _dotb(l_ref[0], wlat_bf)
        cent = (lax.broadcasted_iota(jnp.int32, (1, nrbf), 1).astype(_F32)
                * (12.0 / (nrbf - 1)))
        wrbf_bf = wrbf_ref[...].astype(_BF)
        w1a_bf = w1a_ref[...].astype(_BF)
        w1b_bf = w1b_ref[...].astype(_BF)
        w2a_bf = w2a_ref[...].astype(_BF)
        w2b_bf = w2b_ref[...].astype(_BF)
        wg_bf = wg_ref[...].astype(_BF)
        we_bf = we_ref[...].astype(_BF)

        eiota = lax.broadcasted_iota(jnp.int32, (epb, aps), 1)
        siota = lax.broadcasted_iota(jnp.int32, (aps, epb), 0)
        P, Qt, dirn, ebf = [], [], [], []
        rbf_parts = []
        for j in range(G):
            srcl = src_ref[0, j] & (aps - 1)
            dstl = dst_ref[0, j] & (aps - 1)
            Pm = eiota == srcl[:, None]
            Qm = eiota == dstl[:, None]
            P.append(Pm.astype(_BF))
            Qt.append((siota == dstl[None, :]).astype(_BF))
            R = (Qm.astype(_F32) - Pm.astype(_F32)).astype(_BF)
            xs = x_ref[0, j * aps:(j + 1) * aps, :]
            vec = _dotx(R, xs)
            d2 = jnp.sum(vec * vec, axis=-1, keepdims=True)
            dist = jnp.sqrt(d2 + 1e-12)
            dirn.append(vec / (dist + 1e-8))
            rbf_parts.append(jnp.exp(-((dist - cent) ** 2) * 2.0))
        ebf_all = _dotb(jnp.concatenate(rbf_parts, axis=0), wrbf_bf)
        ebf = [ebf_all[j * epb:(j + 1) * epb] for j in range(G)]
        h = jnp.concatenate(
            [h0_ref[0, j * aps:(j + 1) * aps, :] + base[j:j + 1]
             for j in range(G)], axis=0)
        for wa_bf, wb_bf in ((w1a_bf, w1b_bf), (w2a_bf, w2b_bf)):
            s = _silu(_dotb(h, wa_bf))
            m = [_dot2(P[j], s[j * aps:(j + 1) * aps]) * ebf[j]
                 for j in range(G)]
            agg = jnp.concatenate(
                [_dot2(Qt[j], m[j]) for j in range(G)], axis=0)
            h = h + _silu(_dotb(agg, wb_bf))
        sg = _silu(h)
        ge = [_dot2(P[j], sg[j * aps:(j + 1) * aps]) * ebf[j]
              for j in range(G)]
        gate_all = _dotb(jnp.concatenate(ge, axis=0), wg_bf)
        for j in range(G):
            posv_ref[0, j * aps:(j + 1) * aps, :] = _dot2(
                Qt[j], dirn[j] * gate_all[j * epb:(j + 1) * epb])
        ea = _dotb(h, we_bf)
        for j in range(G):
            pe_ref[0, j, :] = jnp.sum(ea[j * aps:(j + 1) * aps], axis=0)
    return body


def kernel(a, l, x, n, t, edge_index, atom_emb, W_t, W_lat, W_rbf,
           W1a, W1b, W2a, W2b, W_gate, W_energy):
    del n
    b = t.shape[0]
    natoms = a.shape[0]
    aps = natoms // b
    e = edge_index.shape[1]
    epb = e // b
    d = atom_emb.shape[1]
    nrbf = W_rbf.shape[0]
    nelem = atom_emb.shape[0]
    nb = b // G

    h0 = _sc_embed_gather(a.astype(jnp.int32), atom_emb)
    h03 = h0.reshape(nb, G * aps, d)
    t3 = t.reshape(nb, G, 1).astype(jnp.int32)
    l3 = l.reshape(b, 9).reshape(nb, G, 9)
    x3 = x.reshape(nb, G * aps, 3)
    src3 = edge_index[0].reshape(nb, G, epb)
    dst3 = edge_index[1].reshape(nb, G, epb)

    full = lambda shape: pl.BlockSpec(shape, lambda i: tuple(0 for _ in shape))
    posv, pe = pl.pallas_call(
        _make_body(aps, epb, d, nrbf, nelem),
        grid=(nb,),
        in_specs=[
            pl.BlockSpec((1, G * aps, d), lambda i: (i, 0, 0)),
            pl.BlockSpec((1, G, 1), lambda i: (i, 0, 0)),
            pl.BlockSpec((1, G, 9), lambda i: (i, 0, 0)),
            pl.BlockSpec((1, G * aps, 3), lambda i: (i, 0, 0)),
            pl.BlockSpec((1, G, epb), lambda i: (i, 0, 0)),
            pl.BlockSpec((1, G, epb), lambda i: (i, 0, 0)),
            full((d, d)),
            full((9, d)),
            full((nrbf, d)),
            full((d, d)),
            full((d, d)),
            full((d, d)),
            full((d, d)),
            full((d, 1)),
            full((d, 1)),
        ],
        out_specs=[
            pl.BlockSpec((1, G * aps, 3), lambda i: (i, 0, 0)),
            pl.BlockSpec((1, G, 1), lambda i: (i, 0, 0)),
        ],
        out_shape=[
            jax.ShapeDtypeStruct((nb, G * aps, 3), jnp.float32),
            jax.ShapeDtypeStruct((nb, G, 1), jnp.float32),
        ],
    )(h03, t3, l3, x3, src3, dst3, W_t, W_lat, W_rbf,
      W1a, W1b, W2a, W2b, W_gate, W_energy)
    return posv.reshape(natoms, 3), pe.reshape(b)


# speedup vs baseline: 2.6273x; 1.0883x over previous
import functools

import jax
import jax.numpy as jnp
import numpy as np
from jax import lax
from jax.experimental import pallas as pl
from jax.experimental.pallas import tpu as pltpu
from jax.experimental.pallas import tpu_sc as plsc

G = 8
_BF = jnp.bfloat16
_F32 = jnp.float32


def _silu(v):
    return v * jax.nn.sigmoid(v)


def _split3(v):
    h1 = v.astype(_BF)
    r1 = v - h1.astype(_F32)
    h2 = r1.astype(_BF)
    h3 = (r1 - h2.astype(_F32)).astype(_BF)
    return h1, h2, h3


def _dotx(oh_bf, v):
    h1, h2, h3 = _split3(v)
    return (jnp.dot(oh_bf, h1, preferred_element_type=_F32)
            + jnp.dot(oh_bf, h2, preferred_element_type=_F32)
            + jnp.dot(oh_bf, h3, preferred_element_type=_F32))


def _dot2(oh_bf, v):
    h1 = v.astype(_BF)
    h2 = (v - h1.astype(_F32)).astype(_BF)
    return (jnp.dot(oh_bf, h1, preferred_element_type=_F32)
            + jnp.dot(oh_bf, h2, preferred_element_type=_F32))


def _dotb(a, b_bf):
    return jnp.dot(a.astype(_BF), b_bf, preferred_element_type=_F32)


def _sc_embed_gather(a_idx, atom_emb):
    n = a_idx.shape[0]
    d = atom_emb.shape[1]
    info = plsc.get_sparse_core_info()
    nw = info.num_cores * info.num_subcores
    ch = 128
    per_w = n // nw
    mesh = plsc.VectorSubcoreMesh(core_axis_name="c", subcore_axis_name="s")

    @functools.partial(
        pl.kernel, mesh=mesh,
        out_type=jax.ShapeDtypeStruct((n, d), jnp.float32),
        scratch_types=[
            pltpu.VMEM((ch,), jnp.int32),
            pltpu.VMEM((ch, d), jnp.float32),
            pltpu.SemaphoreType.DMA,
        ],
    )
    def k(idx_hbm, table_hbm, out_hbm, idx_v, rows_v, sem):
        wid = lax.axis_index("s") * info.num_cores + lax.axis_index("c")
        base = wid * per_w
        for c in range(per_w // ch):
            off = base + c * ch
            pltpu.sync_copy(idx_hbm.at[pl.ds(off, ch)], idx_v)
            pltpu.async_copy(table_hbm.at[idx_v], rows_v, sem).wait()
            pltpu.sync_copy(rows_v, out_hbm.at[pl.ds(off, ch)])

    return k(a_idx, atom_emb)


def _make_body(aps, epb, d, nrbf, nelem):
    del nelem
    def body(h0_ref, t_ref, l_ref, x_ref, src_ref, dst_ref,
             wt_ref, wlat_ref, wrbf_ref, w1a_ref, w1b_ref,
             w2a_ref, w2b_ref, wg_ref, we_ref,
             posv_ref, pe_ref):
        half = d // 2
        freqs = jnp.exp(
            lax.broadcasted_iota(jnp.int32, (1, half), 1).astype(_F32)
            * (-np.log(10000.0) / half))
        tv = t_ref[0].astype(_F32)
        args = tv * freqs
        temb = jnp.concatenate([jnp.sin(args), jnp.cos(args)], axis=-1)
        wt_bf = wt_ref[...].astype(_BF)
        wlat_bf = wlat_ref[...].astype(_BF)
        base = _dotb(temb, wt_bf) + _dotb(l_ref[0], wlat_bf)
        cent = (lax.broadcasted_iota(jnp.int32, (1, nrbf), 1).astype(_F32)
                * (12.0 / (nrbf - 1)))
        wrbf_bf = wrbf_ref[...].astype(_BF)
        w1a_bf = w1a_ref[...].astype(_BF)
        w1b_bf = w1b_ref[...].astype(_BF)
        w2a_bf = w2a_ref[...].astype(_BF)
        w2b_bf = w2b_ref[...].astype(_BF)
        wg_bf = wg_ref[...].astype(_BF)
        we_bf = we_ref[...].astype(_BF)

        eiota = lax.broadcasted_iota(jnp.int32, (epb, aps), 1)
        siota = lax.broadcasted_iota(jnp.int32, (aps, epb), 0)
        P, Qt, dirn, ebf = [], [], [], []
        rbf_parts = []
        for j in range(G):
            srcl = src_ref[0, j] & (aps - 1)
            dstl = dst_ref[0, j] & (aps - 1)
            Pm = eiota == srcl[:, None]
            Qm = eiota == dstl[:, None]
            P.append(Pm.astype(_BF))
            Qt.append((siota == dstl[None, :]).astype(_BF))
            R = (Qm.astype(_F32) - Pm.astype(_F32)).astype(_BF)
            xs = x_ref[0, j * aps:(j + 1) * aps, :]
            vec = _dotx(R, xs)
            d2 = jnp.sum(vec * vec, axis=-1, keepdims=True)
            dist = jnp.sqrt(d2 + 1e-12)
            dirn.append(vec / (dist + 1e-8))
            rbf_parts.append(jnp.exp(-((dist - cent) ** 2) * 2.0))
        ebf_all = _dotb(jnp.concatenate(rbf_parts, axis=0), wrbf_bf)
        ebf = [ebf_all[j * epb:(j + 1) * epb] for j in range(G)]
        h = jnp.concatenate(
            [h0_ref[0, j * aps:(j + 1) * aps, :] + base[j:j + 1]
             for j in range(G)], axis=0)
        for wa_bf, wb_bf in ((w1a_bf, w1b_bf), (w2a_bf, w2b_bf)):
            s = _silu(_dotb(h, wa_bf))
            m = [_dot2(P[j], s[j * aps:(j + 1) * aps]) * ebf[j]
                 for j in range(G)]
            agg = jnp.concatenate(
                [_dot2(Qt[j], m[j]) for j in range(G)], axis=0)
            h = h + _silu(_dotb(agg, wb_bf))
        sg = _silu(h)
        ge = [_dot2(P[j], sg[j * aps:(j + 1) * aps]) * ebf[j]
              for j in range(G)]
        gate_all = _dotb(jnp.concatenate(ge, axis=0), wg_bf)
        for j in range(G):
            posv_ref[0, j * aps:(j + 1) * aps, :] = _dot2(
                Qt[j], dirn[j] * gate_all[j * epb:(j + 1) * epb])
        ea = _dotb(h, we_bf)
        for j in range(G):
            pe_ref[0, j, :] = jnp.sum(ea[j * aps:(j + 1) * aps], axis=0)
    return body


def kernel(a, l, x, n, t, edge_index, atom_emb, W_t, W_lat, W_rbf,
           W1a, W1b, W2a, W2b, W_gate, W_energy):
    del n
    b = t.shape[0]
    natoms = a.shape[0]
    aps = natoms // b
    e = edge_index.shape[1]
    epb = e // b
    d = atom_emb.shape[1]
    nrbf = W_rbf.shape[0]
    nelem = atom_emb.shape[0]
    nb = b // G

    h0 = _sc_embed_gather(a.astype(jnp.int32), atom_emb)
    h03 = h0.reshape(nb, G * aps, d)
    t3 = t.reshape(nb, G, 1).astype(jnp.int32)
    l3 = l.reshape(b, 9).reshape(nb, G, 9)
    x3 = x.reshape(nb, G * aps, 3)
    src3 = edge_index[0].reshape(nb, G, epb)
    dst3 = edge_index[1].reshape(nb, G, epb)

    full = lambda shape: pl.BlockSpec(shape, lambda i: tuple(0 for _ in shape))
    posv, pe = pl.pallas_call(
        _make_body(aps, epb, d, nrbf, nelem),
        grid=(nb,),
        in_specs=[
            pl.BlockSpec((1, G * aps, d), lambda i: (i, 0, 0)),
            pl.BlockSpec((1, G, 1), lambda i: (i, 0, 0)),
            pl.BlockSpec((1, G, 9), lambda i: (i, 0, 0)),
            pl.BlockSpec((1, G * aps, 3), lambda i: (i, 0, 0)),
            pl.BlockSpec((1, G, epb), lambda i: (i, 0, 0)),
            pl.BlockSpec((1, G, epb), lambda i: (i, 0, 0)),
            full((d, d)),
            full((9, d)),
            full((nrbf, d)),
            full((d, d)),
            full((d, d)),
            full((d, d)),
            full((d, d)),
            full((d, 1)),
            full((d, 1)),
        ],
        out_specs=[
            pl.BlockSpec((1, G * aps, 3), lambda i: (i, 0, 0)),
            pl.BlockSpec((1, G, 1), lambda i: (i, 0, 0)),
        ],
        out_shape=[
            jax.ShapeDtypeStruct((nb, G * aps, 3), jnp.float32),
            jax.ShapeDtypeStruct((nb, G, 1), jnp.float32),
        ],
    )(h03, t3, l3, x3, src3, dst3, W_t, W_lat, W_rbf,
      W1a, W1b, W2a, W2b, W_gate, W_energy)
    return posv.reshape(natoms, 3), pe.reshape(b)


# speedup vs baseline: 2.7232x; 1.0365x over previous
import functools

import jax
import jax.numpy as jnp
import numpy as np
from jax import lax
from jax.experimental import pallas as pl
from jax.experimental.pallas import tpu as pltpu
from jax.experimental.pallas import tpu_sc as plsc

G = 16
_BF = jnp.bfloat16
_F32 = jnp.float32


def _silu(v):
    return v * jax.nn.sigmoid(v)


def _split3(v):
    h1 = v.astype(_BF)
    r1 = v - h1.astype(_F32)
    h2 = r1.astype(_BF)
    h3 = (r1 - h2.astype(_F32)).astype(_BF)
    return h1, h2, h3


def _dotx(oh_bf, v):
    h1, h2, h3 = _split3(v)
    return (jnp.dot(oh_bf, h1, preferred_element_type=_F32)
            + jnp.dot(oh_bf, h2, preferred_element_type=_F32)
            + jnp.dot(oh_bf, h3, preferred_element_type=_F32))


def _dot2(oh_bf, v):
    h1 = v.astype(_BF)
    h2 = (v - h1.astype(_F32)).astype(_BF)
    return (jnp.dot(oh_bf, h1, preferred_element_type=_F32)
            + jnp.dot(oh_bf, h2, preferred_element_type=_F32))


def _dotb(a, b_bf):
    return jnp.dot(a.astype(_BF), b_bf, preferred_element_type=_F32)


def _sc_embed_gather(a_idx, atom_emb):
    n = a_idx.shape[0]
    d = atom_emb.shape[1]
    info = plsc.get_sparse_core_info()
    nw = info.num_cores * info.num_subcores
    ch = 128
    per_w = n // nw
    mesh = plsc.VectorSubcoreMesh(core_axis_name="c", subcore_axis_name="s")

    @functools.partial(
        pl.kernel, mesh=mesh,
        out_type=jax.ShapeDtypeStruct((n, d), jnp.float32),
        scratch_types=[
            pltpu.VMEM((ch,), jnp.int32),
            pltpu.VMEM((ch, d), jnp.float32),
            pltpu.SemaphoreType.DMA,
        ],
    )
    def k(idx_hbm, table_hbm, out_hbm, idx_v, rows_v, sem):
        wid = lax.axis_index("s") * info.num_cores + lax.axis_index("c")
        base = wid * per_w
        for c in range(per_w // ch):
            off = base + c * ch
            pltpu.sync_copy(idx_hbm.at[pl.ds(off, ch)], idx_v)
            pltpu.async_copy(table_hbm.at[idx_v], rows_v, sem).wait()
            pltpu.sync_copy(rows_v, out_hbm.at[pl.ds(off, ch)])

    return k(a_idx, atom_emb)


def _make_body(aps, epb, d, nrbf, nelem):
    del nelem
    def body(h0_ref, t_ref, l_ref, x_ref, src_ref, dst_ref,
             wt_ref, wlat_ref, wrbf_ref, w1a_ref, w1b_ref,
             w2a_ref, w2b_ref, wg_ref, we_ref,
             posv_ref, pe_ref):
        half = d // 2
        freqs = jnp.exp(
            lax.broadcasted_iota(jnp.int32, (1, half), 1).astype(_F32)
            * (-np.log(10000.0) / half))
        tv = t_ref[0].astype(_F32)
        args = tv * freqs
        temb = jnp.concatenate([jnp.sin(args), jnp.cos(args)], axis=-1)
        wt_bf = wt_ref[...].astype(_BF)
        wlat_bf = wlat_ref[...].astype(_BF)
        base = _dotb(temb, wt_bf) + _dotb(l_ref[0], wlat_bf)
        cent = (lax.broadcasted_iota(jnp.int32, (1, nrbf), 1).astype(_F32)
                * (12.0 / (nrbf - 1)))
        wrbf_bf = wrbf_ref[...].astype(_BF)
        w1a_bf = w1a_ref[...].astype(_BF)
        w1b_bf = w1b_ref[...].astype(_BF)
        w2a_bf = w2a_ref[...].astype(_BF)
        w2b_bf = w2b_ref[...].astype(_BF)
        wg_bf = wg_ref[...].astype(_BF)
        we_bf = we_ref[...].astype(_BF)

        eiota = lax.broadcasted_iota(jnp.int32, (epb, aps), 1)
        siota = lax.broadcasted_iota(jnp.int32, (aps, epb), 0)
        P, Qt, dirn, ebf = [], [], [], []
        rbf_parts = []
        for j in range(G):
            srcl = src_ref[0, j] & (aps - 1)
            dstl = dst_ref[0, j] & (aps - 1)
            Pm = eiota == srcl[:, None]
            Qm = eiota == dstl[:, None]
            P.append(Pm.astype(_BF))
            Qt.append((siota == dstl[None, :]).astype(_BF))
            R = (Qm.astype(_F32) - Pm.astype(_F32)).astype(_BF)
            xs = x_ref[0, j * aps:(j + 1) * aps, :]
            vec = _dotx(R, xs)
            d2 = jnp.sum(vec * vec, axis=-1, keepdims=True)
            dist = jnp.sqrt(d2 + 1e-12)
            dirn.append(vec / (dist + 1e-8))
            rbf_parts.append(jnp.exp(-((dist - cent) ** 2) * 2.0))
        ebf_all = _dotb(jnp.concatenate(rbf_parts, axis=0), wrbf_bf)
        ebf = [ebf_all[j * epb:(j + 1) * epb] for j in range(G)]
        h = jnp.concatenate(
            [h0_ref[0, j * aps:(j + 1) * aps, :] + base[j:j + 1]
             for j in range(G)], axis=0)
        for wa_bf, wb_bf in ((w1a_bf, w1b_bf), (w2a_bf, w2b_bf)):
            s = _silu(_dotb(h, wa_bf))
            m = [_dot2(P[j], s[j * aps:(j + 1) * aps]) * ebf[j]
                 for j in range(G)]
            agg = jnp.concatenate(
                [_dot2(Qt[j], m[j]) for j in range(G)], axis=0)
            h = h + _silu(_dotb(agg, wb_bf))
        sg = _silu(h)
        ge = [_dot2(P[j], sg[j * aps:(j + 1) * aps]) * ebf[j]
              for j in range(G)]
        gate_all = _dotb(jnp.concatenate(ge, axis=0), wg_bf)
        for j in range(G):
            posv_ref[0, j * aps:(j + 1) * aps, :] = _dot2(
                Qt[j], dirn[j] * gate_all[j * epb:(j + 1) * epb])
        ea = _dotb(h, we_bf)
        for j in range(G):
            pe_ref[0, j, :] = jnp.sum(ea[j * aps:(j + 1) * aps], axis=0)
    return body


def kernel(a, l, x, n, t, edge_index, atom_emb, W_t, W_lat, W_rbf,
           W1a, W1b, W2a, W2b, W_gate, W_energy):
    del n
    b = t.shape[0]
    natoms = a.shape[0]
    aps = natoms // b
    e = edge_index.shape[1]
    epb = e // b
    d = atom_emb.shape[1]
    nrbf = W_rbf.shape[0]
    nelem = atom_emb.shape[0]
    nb = b // G

    h0 = _sc_embed_gather(a.astype(jnp.int32), atom_emb)
    h03 = h0.reshape(nb, G * aps, d)
    t3 = t.reshape(nb, G, 1).astype(jnp.int32)
    l3 = l.reshape(b, 9).reshape(nb, G, 9)
    x3 = x.reshape(nb, G * aps, 3)
    src3 = edge_index[0].reshape(nb, G, epb)
    dst3 = edge_index[1].reshape(nb, G, epb)

    full = lambda shape: pl.BlockSpec(shape, lambda i: tuple(0 for _ in shape))
    posv, pe = pl.pallas_call(
        _make_body(aps, epb, d, nrbf, nelem),
        grid=(nb,),
        in_specs=[
            pl.BlockSpec((1, G * aps, d), lambda i: (i, 0, 0)),
            pl.BlockSpec((1, G, 1), lambda i: (i, 0, 0)),
            pl.BlockSpec((1, G, 9), lambda i: (i, 0, 0)),
            pl.BlockSpec((1, G * aps, 3), lambda i: (i, 0, 0)),
            pl.BlockSpec((1, G, epb), lambda i: (i, 0, 0)),
            pl.BlockSpec((1, G, epb), lambda i: (i, 0, 0)),
            full((d, d)),
            full((9, d)),
            full((nrbf, d)),
            full((d, d)),
            full((d, d)),
            full((d, d)),
            full((d, d)),
            full((d, 1)),
            full((d, 1)),
        ],
        out_specs=[
            pl.BlockSpec((1, G * aps, 3), lambda i: (i, 0, 0)),
            pl.BlockSpec((1, G, 1), lambda i: (i, 0, 0)),
        ],
        out_shape=[
            jax.ShapeDtypeStruct((nb, G * aps, 3), jnp.float32),
            jax.ShapeDtypeStruct((nb, G, 1), jnp.float32),
        ],
    )(h03, t3, l3, x3, src3, dst3, W_t, W_lat, W_rbf,
      W1a, W1b, W2a, W2b, W_gate, W_energy)
    return posv.reshape(natoms, 3), pe.reshape(b)


# speedup vs baseline: 2.7319x; 1.0032x over previous
import functools

import jax
import jax.numpy as jnp
import numpy as np
from jax import lax
from jax.experimental import pallas as pl
from jax.experimental.pallas import tpu as pltpu
from jax.experimental.pallas import tpu_sc as plsc

G = 16
_BF = jnp.bfloat16
_F32 = jnp.float32


def _silu(v):
    return v * jax.nn.sigmoid(v)


def _split3(v):
    h1 = v.astype(_BF)
    r1 = v - h1.astype(_F32)
    h2 = r1.astype(_BF)
    h3 = (r1 - h2.astype(_F32)).astype(_BF)
    return h1, h2, h3


def _dotx(oh_bf, v):
    h1, h2, h3 = _split3(v)
    return (jnp.dot(oh_bf, h1, preferred_element_type=_F32)
            + jnp.dot(oh_bf, h2, preferred_element_type=_F32)
            + jnp.dot(oh_bf, h3, preferred_element_type=_F32))


def _dot2(oh_bf, v):
    h1 = v.astype(_BF)
    h2 = (v - h1.astype(_F32)).astype(_BF)
    return (jnp.dot(oh_bf, h1, preferred_element_type=_F32)
            + jnp.dot(oh_bf, h2, preferred_element_type=_F32))


def _dotb(a, b_bf):
    return jnp.dot(a.astype(_BF), b_bf, preferred_element_type=_F32)


def _sc_embed_gather(a_idx, atom_emb):
    n = a_idx.shape[0]
    d = atom_emb.shape[1]
    info = plsc.get_sparse_core_info()
    nw = info.num_cores * info.num_subcores
    ch = 128
    per_w = n // nw
    mesh = plsc.VectorSubcoreMesh(core_axis_name="c", subcore_axis_name="s")

    nch = per_w // ch

    @functools.partial(
        pl.kernel, mesh=mesh,
        out_type=jax.ShapeDtypeStruct((n, d), jnp.float32),
        scratch_types=[
            pltpu.VMEM((per_w,), jnp.int32),
            pltpu.VMEM((2, ch, d), jnp.float32),
            pltpu.SemaphoreType.DMA,
            pltpu.SemaphoreType.DMA,
            pltpu.SemaphoreType.DMA,
        ],
    )
    def k(idx_hbm, table_hbm, out_hbm, idx_v, rows_v, g0, g1, wsem):
        wid = lax.axis_index("s") * info.num_cores + lax.axis_index("c")
        base = wid * per_w
        gsem = (g0, g1)
        pltpu.sync_copy(idx_hbm.at[pl.ds(base, per_w)], idx_v)
        pend = [None, None]
        pend[0] = pltpu.async_copy(
            table_hbm.at[idx_v.at[pl.ds(0, ch)]], rows_v.at[0], g0)
        for c in range(nch):
            s = c % 2
            if c + 1 < nch:
                pend[1 - s] = pltpu.async_copy(
                    table_hbm.at[idx_v.at[pl.ds((c + 1) * ch, ch)]],
                    rows_v.at[1 - s], gsem[1 - s])
            pend[s].wait()
            pltpu.async_copy(
                rows_v.at[s], out_hbm.at[pl.ds(base + c * ch, ch)],
                wsem).wait()

    return k(a_idx, atom_emb)


def _make_body(aps, epb, d, nrbf, nelem):
    del nelem
    def body(h0_ref, t_ref, l_ref, x_ref, src_ref, dst_ref,
             wt_ref, wlat_ref, wrbf_ref, w1a_ref, w1b_ref,
             w2a_ref, w2b_ref, wg_ref, we_ref,
             posv_ref, pe_ref):
        half = d // 2
        freqs = jnp.exp(
            lax.broadcasted_iota(jnp.int32, (1, half), 1).astype(_F32)
            * (-np.log(10000.0) / half))
        tv = t_ref[0].astype(_F32)
        args = tv * freqs
        temb = jnp.concatenate([jnp.sin(args), jnp.cos(args)], axis=-1)
        wt_bf = wt_ref[...].astype(_BF)
        wlat_bf = wlat_ref[...].astype(_BF)
        base = _dotb(temb, wt_bf) + _dotb(l_ref[0], wlat_bf)
        cent = (lax.broadcasted_iota(jnp.int32, (1, nrbf), 1).astype(_F32)
                * (12.0 / (nrbf - 1)))
        wrbf_bf = wrbf_ref[...].astype(_BF)
        w1a_bf = w1a_ref[...].astype(_BF)
        w1b_bf = w1b_ref[...].astype(_BF)
        w2a_bf = w2a_ref[...].astype(_BF)
        w2b_bf = w2b_ref[...].astype(_BF)
        wg_bf = wg_ref[...].astype(_BF)
        we_bf = we_ref[...].astype(_BF)

        eiota = lax.broadcasted_iota(jnp.int32, (epb, aps), 1)
        siota = lax.broadcasted_iota(jnp.int32, (aps, epb), 0)
        P, Qt, dirn, ebf = [], [], [], []
        rbf_parts = []
        for j in range(G):
            srcl = src_ref[0, j] & (aps - 1)
            dstl = dst_ref[0, j] & (aps - 1)
            Pm = eiota == srcl[:, None]
            Qm = eiota == dstl[:, None]
            P.append(Pm.astype(_BF))
            Qt.append((siota == dstl[None, :]).astype(_BF))
            R = (Qm.astype(_F32) - Pm.astype(_F32)).astype(_BF)
            xs = x_ref[0, j * aps:(j + 1) * aps, :]
            vec = _dotx(R, xs)
            d2 = jnp.sum(vec * vec, axis=-1, keepdims=True)
            dist = jnp.sqrt(d2 + 1e-12)
            dirn.append(vec / (dist + 1e-8))
            rbf_parts.append(jnp.exp(-((dist - cent) ** 2) * 2.0))
        ebf_all = _dotb(jnp.concatenate(rbf_parts, axis=0), wrbf_bf)
        ebf = [ebf_all[j * epb:(j + 1) * epb] for j in range(G)]
        h = jnp.concatenate(
            [h0_ref[0, j * aps:(j + 1) * aps, :] + base[j:j + 1]
             for j in range(G)], axis=0)
        for wa_bf, wb_bf in ((w1a_bf, w1b_bf), (w2a_bf, w2b_bf)):
            s = _silu(_dotb(h, wa_bf))
            m = [_dot2(P[j], s[j * aps:(j + 1) * aps]) * ebf[j]
                 for j in range(G)]
            agg = jnp.concatenate(
                [_dot2(Qt[j], m[j]) for j in range(G)], axis=0)
            h = h + _silu(_dotb(agg, wb_bf))
        sg = _silu(h)
        ge = [_dot2(P[j], sg[j * aps:(j + 1) * aps]) * ebf[j]
              for j in range(G)]
        gate_all = _dotb(jnp.concatenate(ge, axis=0), wg_bf)
        for j in range(G):
            posv_ref[0, j * aps:(j + 1) * aps, :] = _dot2(
                Qt[j], dirn[j] * gate_all[j * epb:(j + 1) * epb])
        ea = _dotb(h, we_bf)
        for j in range(G):
            pe_ref[0, j, :] = jnp.sum(ea[j * aps:(j + 1) * aps], axis=0)
    return body


def kernel(a, l, x, n, t, edge_index, atom_emb, W_t, W_lat, W_rbf,
           W1a, W1b, W2a, W2b, W_gate, W_energy):
    del n
    b = t.shape[0]
    natoms = a.shape[0]
    aps = natoms // b
    e = edge_index.shape[1]
    epb = e // b
    d = atom_emb.shape[1]
    nrbf = W_rbf.shape[0]
    nelem = atom_emb.shape[0]
    nb = b // G

    h0 = _sc_embed_gather(a.astype(jnp.int32), atom_emb)
    h03 = h0.reshape(nb, G * aps, d)
    t3 = t.reshape(nb, G, 1).astype(jnp.int32)
    l3 = l.reshape(b, 9).reshape(nb, G, 9)
    x3 = x.reshape(nb, G * aps, 3)
    src3 = edge_index[0].reshape(nb, G, epb)
    dst3 = edge_index[1].reshape(nb, G, epb)

    full = lambda shape: pl.BlockSpec(shape, lambda i: tuple(0 for _ in shape))
    posv, pe = pl.pallas_call(
        _make_body(aps, epb, d, nrbf, nelem),
        grid=(nb,),
        in_specs=[
            pl.BlockSpec((1, G * aps, d), lambda i: (i, 0, 0)),
            pl.BlockSpec((1, G, 1), lambda i: (i, 0, 0)),
            pl.BlockSpec((1, G, 9), lambda i: (i, 0, 0)),
            pl.BlockSpec((1, G * aps, 3), lambda i: (i, 0, 0)),
            pl.BlockSpec((1, G, epb), lambda i: (i, 0, 0)),
            pl.BlockSpec((1, G, epb), lambda i: (i, 0, 0)),
            full((d, d)),
            full((9, d)),
            full((nrbf, d)),
            full((d, d)),
            full((d, d)),
            full((d, d)),
            full((d, d)),
            full((d, 1)),
            full((d, 1)),
        ],
        out_specs=[
            pl.BlockSpec((1, G * aps, 3), lambda i: (i, 0, 0)),
            pl.BlockSpec((1, G, 1), lambda i: (i, 0, 0)),
        ],
        out_shape=[
            jax.ShapeDtypeStruct((nb, G * aps, 3), jnp.float32),
            jax.ShapeDtypeStruct((nb, G, 1), jnp.float32),
        ],
    )(h03, t3, l3, x3, src3, dst3, W_t, W_lat, W_rbf,
      W1a, W1b, W2a, W2b, W_gate, W_energy)
    return posv.reshape(natoms, 3), pe.reshape(b)
